# Initial kernel scaffold; baseline (speedup 1.0000x reference)
import jax
import jax.numpy as jnp
from jax.experimental import pallas as pl


def kernel(seq, pbs_feat, rt_feat, nucl_table, pbs_table, rt_table):
    raise NotImplementedError("write your pallas kernel here")


# speedup vs baseline: 2.3865x; 2.3865x over previous
import functools

import jax
import jax.numpy as jnp
from jax import lax
from jax.experimental import pallas as pl
from jax.experimental.pallas import tpu as pltpu
from jax.experimental.pallas import tpu_sc as plsc

B, L = 1024, 200
NUCL_DIM, SPEC_DIM = 128, 64
OUT_DIM = NUCL_DIM + 2 * SPEC_DIM
NW = 32
BPW = B // NW
LPAD = 208
CHUNK_A, CHUNK_B = 128, LPAD - 128


def _body(seq_ref, pbsf_ref, rtf_ref, nucl_ref, pbst_ref, rtt_ref,
          out_ref, ctab_hbm,
          nucl_v, pbst_v, rtt_v, ctab_v, pbsf_v, rtf_v,
          seq_v, idx_a, idx_b, row_buf, sem):
    wid = lax.axis_index("s") * 2 + lax.axis_index("c")
    base = wid * BPW

    pltpu.sync_copy(nucl_ref, nucl_v)
    pltpu.sync_copy(pbst_ref, pbst_v)
    pltpu.sync_copy(rtt_ref, rtt_v)

    for pi in range(2):
        for ri in range(2):
            for v in range(6):
                row = 12 * pi + 6 * ri + v
                for k in range(NUCL_DIM // 16):
                    ctab_v[row, pl.ds(16 * k, 16)] = nucl_v[v, pl.ds(16 * k, 16)]
                for k in range(SPEC_DIM // 16):
                    ctab_v[row, pl.ds(NUCL_DIM + 16 * k, 16)] = pbst_v[pi, pl.ds(16 * k, 16)]
                for k in range(SPEC_DIM // 16):
                    ctab_v[row, pl.ds(NUCL_DIM + SPEC_DIM + 16 * k, 16)] = rtt_v[ri, pl.ds(16 * k, 16)]
    pltpu.sync_copy(ctab_v, ctab_hbm.at[pl.ds(wid * 24, 24)])

    pltpu.sync_copy(pbsf_ref.at[pl.ds(base, BPW)], pbsf_v)
    pltpu.sync_copy(rtf_ref.at[pl.ds(base, BPW)], rtf_v)
    half = jnp.full((16,), 0.5, jnp.float32)
    combos = []
    for k in range(BPW // 16):
        pv = pbsf_v[pl.ds(16 * k, 16)]
        rv = rtf_v[pl.ds(16 * k, 16)]
        combo = jnp.where(pv > half, jnp.int32(12), jnp.int32(0))
        combo = combo + jnp.where(rv > half, jnp.int32(6), jnp.int32(0))
        combos.append(combo + wid * 24)
    lane_ids = lax.iota(jnp.int32, 16)

    seq_v[pl.ds(LPAD - 16, 16)] = jnp.zeros((16,), jnp.int32)

    def batch_step(j, carry):
        g = base + j
        pltpu.sync_copy(seq_ref.at[pl.ds(g * L, L)], seq_v.at[pl.ds(0, L)])
        in_lo = jnp.full((16,), j < 16)
        cvec = jnp.where(in_lo, combos[0], combos[1])
        sel = lane_ids == (j % 16)
        off = jnp.sum(jnp.where(sel, cvec, jnp.int32(0)))
        for k in range(CHUNK_A // 16):
            idx_a[pl.ds(16 * k, 16)] = seq_v[pl.ds(16 * k, 16)] + off
        for k in range(CHUNK_B // 16):
            idx_b[pl.ds(16 * k, 16)] = seq_v[pl.ds(CHUNK_A + 16 * k, 16)] + off
        cp1 = pltpu.async_copy(ctab_hbm.at[idx_a], row_buf.at[pl.ds(0, CHUNK_A)], sem)
        cp2 = pltpu.async_copy(ctab_hbm.at[idx_b], row_buf.at[pl.ds(CHUNK_A, CHUNK_B)], sem)
        cp1.wait()
        cp2.wait()
        pltpu.sync_copy(row_buf.at[pl.ds(0, L)], out_ref.at[pl.ds(g * L, L)])
        return carry

    lax.fori_loop(0, BPW, batch_step, 0)


def kernel(seq, pbs_feat, rt_feat, nucl_table, pbs_table, rt_table):
    mesh = plsc.VectorSubcoreMesh(core_axis_name="c", subcore_axis_name="s")
    run = functools.partial(
        pl.kernel,
        mesh=mesh,
        compiler_params=pltpu.CompilerParams(needs_layout_passes=False),
        out_type=[
            jax.ShapeDtypeStruct((B * L, OUT_DIM), jnp.float32),
            jax.ShapeDtypeStruct((NW * 24, OUT_DIM), jnp.float32),
        ],
        scratch_types=[
            pltpu.VMEM((6, NUCL_DIM), jnp.float32),
            pltpu.VMEM((2, SPEC_DIM), jnp.float32),
            pltpu.VMEM((2, SPEC_DIM), jnp.float32),
            pltpu.VMEM((24, OUT_DIM), jnp.float32),
            pltpu.VMEM((BPW,), jnp.float32),
            pltpu.VMEM((BPW,), jnp.float32),
            pltpu.VMEM((LPAD,), jnp.int32),
            pltpu.VMEM((CHUNK_A,), jnp.int32),
            pltpu.VMEM((CHUNK_B,), jnp.int32),
            pltpu.VMEM((LPAD, OUT_DIM), jnp.float32),
            pltpu.SemaphoreType.DMA,
        ],
    )(_body)
    out, _ = run(seq.reshape(B * L), pbs_feat, rt_feat,
                 nucl_table, pbs_table, rt_table)
    return out.reshape(B, L, OUT_DIM)


# speedup vs baseline: 3.1474x; 1.3189x over previous
import functools

import jax
import jax.numpy as jnp
from jax import lax
from jax.experimental import pallas as pl
from jax.experimental.pallas import tpu as pltpu
from jax.experimental.pallas import tpu_sc as plsc

B, L = 1024, 200
NUCL_DIM, SPEC_DIM = 128, 64
OUT_DIM = NUCL_DIM + 2 * SPEC_DIM
NW = 32
BPW = B // NW
LPAD = 208
CHUNK_A, CHUNK_B = 128, L - 128


def _body(seq_ref, pbsf_ref, rtf_ref, nucl_ref, pbst_ref, rtt_ref,
          out_ref, ctab_hbm,
          nucl_v, pbst_v, rtt_v, ctab_v, pbsf_v, rtf_v,
          seq_all, idx_all, rb0, rb1, sg0, sg1, so0, so1):
    sid = lax.axis_index("s")
    wid = sid * 2 + lax.axis_index("c")
    base = wid * BPW

    pltpu.sync_copy(nucl_ref, nucl_v)
    pltpu.sync_copy(pbst_ref, pbst_v)
    pltpu.sync_copy(rtt_ref, rtt_v)
    for pi in range(2):
        for ri in range(2):
            for v in range(6):
                row = 12 * pi + 6 * ri + v
                for k in range(NUCL_DIM // 16):
                    ctab_v[row, pl.ds(16 * k, 16)] = nucl_v[v, pl.ds(16 * k, 16)]
                for k in range(SPEC_DIM // 16):
                    ctab_v[row, pl.ds(NUCL_DIM + 16 * k, 16)] = pbst_v[pi, pl.ds(16 * k, 16)]
                for k in range(SPEC_DIM // 16):
                    ctab_v[row, pl.ds(NUCL_DIM + SPEC_DIM + 16 * k, 16)] = rtt_v[ri, pl.ds(16 * k, 16)]
    pltpu.sync_copy(ctab_v, ctab_hbm.at[pl.ds(wid * 24, 24)])

    pltpu.sync_copy(pbsf_ref.at[pl.ds(base, BPW)], pbsf_v)
    pltpu.sync_copy(rtf_ref.at[pl.ds(base, BPW)], rtf_v)
    half = jnp.full((16,), 0.5, jnp.float32)
    combos = []
    for k in range(BPW // 16):
        pv = pbsf_v[pl.ds(16 * k, 16)]
        rv = rtf_v[pl.ds(16 * k, 16)]
        combo = jnp.where(pv > half, jnp.int32(12), jnp.int32(0))
        combo = combo + jnp.where(rv > half, jnp.int32(6), jnp.int32(0))
        combos.append(combo + wid * 24)
    lane_ids = lax.iota(jnp.int32, 16)

    seq_all[pl.ds(BPW * L, 16)] = jnp.zeros((16,), jnp.int32)
    pltpu.sync_copy(seq_ref.at[pl.ds(base * L, BPW * L)], seq_all.at[pl.ds(0, BPW * L)])

    for j in range(BPW):
        sel = lane_ids == (j % 16)
        off = jnp.sum(jnp.where(sel, combos[j // 16], jnp.int32(0)))
        for k in range(LPAD // 16):
            idx_all[j, pl.ds(16 * k, 16)] = seq_all[pl.ds(j * L + 16 * k, 16)] + off

    rbs, sgs, sos = (rb0, rb1), (sg0, sg1), (so0, so1)
    g_handles = [None] * BPW
    o_handles = [None] * BPW
    for j in range(BPW):
        if j >= 2:
            o_handles[j - 2].wait()
        rb = rbs[j % 2]
        g_handles[j] = (
            pltpu.async_copy(ctab_hbm.at[idx_all.at[j, pl.ds(0, CHUNK_A)]],
                             rb.at[pl.ds(0, CHUNK_A)], sgs[j % 2]),
            pltpu.async_copy(ctab_hbm.at[idx_all.at[j, pl.ds(CHUNK_A, CHUNK_B)]],
                             rb.at[pl.ds(CHUNK_A, CHUNK_B)], sgs[j % 2]),
        )
        if j >= 1:
            for h in g_handles[j - 1]:
                h.wait()
            o_handles[j - 1] = pltpu.async_copy(
                rbs[(j - 1) % 2], out_ref.at[pl.ds((base + j - 1) * L, L)],
                sos[(j - 1) % 2])
    for h in g_handles[BPW - 1]:
        h.wait()
    o_handles[BPW - 1] = pltpu.async_copy(
        rbs[(BPW - 1) % 2], out_ref.at[pl.ds((base + BPW - 1) * L, L)],
        sos[(BPW - 1) % 2])
    o_handles[BPW - 2].wait()
    o_handles[BPW - 1].wait()


def kernel(seq, pbs_feat, rt_feat, nucl_table, pbs_table, rt_table):
    mesh = plsc.VectorSubcoreMesh(core_axis_name="c", subcore_axis_name="s")
    run = functools.partial(
        pl.kernel,
        mesh=mesh,
        compiler_params=pltpu.CompilerParams(needs_layout_passes=False),
        out_type=[
            jax.ShapeDtypeStruct((B * L, OUT_DIM), jnp.float32),
            jax.ShapeDtypeStruct((NW * 24, OUT_DIM), jnp.float32),
        ],
        scratch_types=[
            pltpu.VMEM((6, NUCL_DIM), jnp.float32),
            pltpu.VMEM((2, SPEC_DIM), jnp.float32),
            pltpu.VMEM((2, SPEC_DIM), jnp.float32),
            pltpu.VMEM((24, OUT_DIM), jnp.float32),
            pltpu.VMEM((BPW,), jnp.float32),
            pltpu.VMEM((BPW,), jnp.float32),
            pltpu.VMEM((BPW * L + 16,), jnp.int32),
            pltpu.VMEM((BPW, LPAD), jnp.int32),
            pltpu.VMEM((L, OUT_DIM), jnp.float32),
            pltpu.VMEM((L, OUT_DIM), jnp.float32),
            pltpu.SemaphoreType.DMA,
            pltpu.SemaphoreType.DMA,
            pltpu.SemaphoreType.DMA,
            pltpu.SemaphoreType.DMA,
        ],
    )(_body)
    out, _ = run(seq.reshape(B * L), pbs_feat, rt_feat,
                 nucl_table, pbs_table, rt_table)
    return out.reshape(B, L, OUT_DIM)
